# Initial kernel scaffold; baseline (speedup 1.0000x reference)
import jax
import jax.numpy as jnp
from jax.experimental import pallas as pl


def kernel(x, W1, b1, W2, b2):
    raise NotImplementedError("write your pallas kernel here")


# speedup vs baseline: 62.5533x; 62.5533x over previous
import functools

import jax
import jax.numpy as jnp
from jax.experimental import pallas as pl

BATCH, V, SAMPLES = 32, 64, 4096
EMBED_DIM = 128
WINDOW_SIZE = 64
STEP_SIZE = 32
NUM_TOKENS = (SAMPLES - WINDOW_SIZE) // STEP_SIZE
NUM_CHUNKS = SAMPLES // STEP_SIZE

_PREC = jax.lax.Precision.HIGHEST


def _encode_kernel(x_ref, w1_ref, b1_ref, out_ref):
    x = x_ref[0]
    proj = jax.lax.dot_general(
        x, w1_ref[...],
        dimension_numbers=(((0,), (0,)), ((), ())),
        precision=_PREC, preferred_element_type=jnp.float32)
    out_ref[0] = jnp.floor(proj + b1_ref[...][None, :])


def _token_kernel(p_ref, wt_ref, wb_ref, b2_ref, out_ref):
    p = p_ref[0]
    u = jnp.dot(p, wt_ref[...], precision=_PREC,
                preferred_element_type=jnp.float32)
    v = jnp.dot(p, wb_ref[...], precision=_PREC,
                preferred_element_type=jnp.float32)
    out_ref[0] = jnp.floor(u[:NUM_TOKENS] + v[1:NUM_TOKENS + 1]
                           + b2_ref[...][None, :])


def _imap3(b):
    z = b - b
    return (b, z, z)


def _imap2(b):
    z = b - b
    return (z, z)


def _imap1(b):
    return (b - b,)


@functools.partial(jax.jit, static_argnames=())
def kernel(x, W1, b1, W2, b2):
    xf = x.astype(jnp.float32)
    w1f = W1.astype(jnp.float32)
    b1f = b1.astype(jnp.float32)
    b2f = b2.astype(jnp.float32)
    w2f = W2.astype(jnp.float32).reshape(EMBED_DIM, WINDOW_SIZE, EMBED_DIM)
    w2p = jnp.transpose(w2f, (1, 0, 2))
    wtop = w2p[:STEP_SIZE].reshape(STEP_SIZE * EMBED_DIM, EMBED_DIM)
    wbot = w2p[STEP_SIZE:].reshape(STEP_SIZE * EMBED_DIM, EMBED_DIM)

    ns = pl.pallas_call(
        _encode_kernel,
        grid=(BATCH,),
        in_specs=[
            pl.BlockSpec((1, V, SAMPLES), _imap3),
            pl.BlockSpec((V, EMBED_DIM), _imap2),
            pl.BlockSpec((EMBED_DIM,), _imap1),
        ],
        out_specs=pl.BlockSpec((1, SAMPLES, EMBED_DIM), _imap3),
        out_shape=jax.ShapeDtypeStruct((BATCH, SAMPLES, EMBED_DIM),
                                       jnp.float32),
    )(xf, w1f, b1f)

    p = ns.reshape(BATCH, NUM_CHUNKS, STEP_SIZE * EMBED_DIM)

    tokens = pl.pallas_call(
        _token_kernel,
        grid=(BATCH,),
        in_specs=[
            pl.BlockSpec((1, NUM_CHUNKS, STEP_SIZE * EMBED_DIM), _imap3),
            pl.BlockSpec((STEP_SIZE * EMBED_DIM, EMBED_DIM), _imap2),
            pl.BlockSpec((STEP_SIZE * EMBED_DIM, EMBED_DIM), _imap2),
            pl.BlockSpec((EMBED_DIM,), _imap1),
        ],
        out_specs=pl.BlockSpec((1, NUM_TOKENS, EMBED_DIM), _imap3),
        out_shape=jax.ShapeDtypeStruct((BATCH, NUM_TOKENS, EMBED_DIM),
                                       jnp.float32),
    )(p, wtop, wbot, b2f)

    return tokens.astype(jnp.float64)


# speedup vs baseline: 90.4252x; 1.4456x over previous
import functools

import jax
import jax.numpy as jnp
from jax.experimental import pallas as pl

BATCH, V, SAMPLES = 32, 64, 4096
EMBED_DIM = 128
WINDOW_SIZE = 64
STEP_SIZE = 32
NUM_TOKENS = (SAMPLES - WINDOW_SIZE) // STEP_SIZE
NUM_CHUNKS = SAMPLES // STEP_SIZE
CHUNK_FLAT = STEP_SIZE * EMBED_DIM

_PREC = jax.lax.Precision.HIGHEST


def _fused_kernel(x_ref, w1_ref, b1_ref, wth_ref, wtl_ref, wbh_ref, wbl_ref,
                  b2_ref, out_ref):
    proj = jax.lax.dot_general(
        x_ref[0], w1_ref[...],
        dimension_numbers=(((0,), (0,)), ((), ())),
        precision=_PREC, preferred_element_type=jnp.float32)
    ns = jnp.floor(proj + b1_ref[...][None, :])
    p = ns.reshape(NUM_CHUNKS, CHUNK_FLAT).astype(jnp.bfloat16)
    u = (jnp.dot(p, wth_ref[...], preferred_element_type=jnp.float32)
         + jnp.dot(p, wtl_ref[...], preferred_element_type=jnp.float32))
    v = (jnp.dot(p, wbh_ref[...], preferred_element_type=jnp.float32)
         + jnp.dot(p, wbl_ref[...], preferred_element_type=jnp.float32))
    out_ref[0] = jnp.floor(u[:NUM_TOKENS] + v[1:NUM_TOKENS + 1]
                           + b2_ref[...][None, :])


def _imap3(b):
    z = b - b
    return (b, z, z)


def _imap2(b):
    z = b - b
    return (z, z)


def _imap1(b):
    return (b - b,)


@functools.partial(jax.jit, static_argnames=())
def kernel(x, W1, b1, W2, b2):
    xf = x.astype(jnp.float32)
    w1f = W1.astype(jnp.float32)
    b1f = b1.astype(jnp.float32)
    b2f = b2.astype(jnp.float32)
    w2f = W2.astype(jnp.float32).reshape(EMBED_DIM, WINDOW_SIZE, EMBED_DIM)
    w2p = jnp.transpose(w2f, (1, 0, 2))
    wtop = w2p[:STEP_SIZE].reshape(CHUNK_FLAT, EMBED_DIM)
    wbot = w2p[STEP_SIZE:].reshape(CHUNK_FLAT, EMBED_DIM)
    wth = wtop.astype(jnp.bfloat16)
    wtl = (wtop - wth.astype(jnp.float32)).astype(jnp.bfloat16)
    wbh = wbot.astype(jnp.bfloat16)
    wbl = (wbot - wbh.astype(jnp.float32)).astype(jnp.bfloat16)

    wspec = pl.BlockSpec((CHUNK_FLAT, EMBED_DIM), _imap2)
    tokens = pl.pallas_call(
        _fused_kernel,
        grid=(BATCH,),
        in_specs=[
            pl.BlockSpec((1, V, SAMPLES), _imap3),
            pl.BlockSpec((V, EMBED_DIM), _imap2),
            pl.BlockSpec((EMBED_DIM,), _imap1),
            wspec, wspec, wspec, wspec,
            pl.BlockSpec((EMBED_DIM,), _imap1),
        ],
        out_specs=pl.BlockSpec((1, NUM_TOKENS, EMBED_DIM), _imap3),
        out_shape=jax.ShapeDtypeStruct((BATCH, NUM_TOKENS, EMBED_DIM),
                                       jnp.float32),
    )(xf, w1f, b1f, wth, wtl, wbh, wbl, b2f)

    return tokens.astype(jnp.float64)


# speedup vs baseline: 123.3739x; 1.3644x over previous
import functools

import jax
import jax.numpy as jnp
from jax.experimental import pallas as pl

BATCH, V, SAMPLES = 32, 64, 4096
EMBED_DIM = 128
WINDOW_SIZE = 64
STEP_SIZE = 32
NUM_TOKENS = (SAMPLES - WINDOW_SIZE) // STEP_SIZE
NUM_CHUNKS = SAMPLES // STEP_SIZE
CHUNK_FLAT = STEP_SIZE * EMBED_DIM

_PREC = jax.lax.Precision.HIGHEST


def _fused_kernel(x_ref, w1_ref, b1_ref, wcat_ref, b2_ref, out_ref):
    proj = jax.lax.dot_general(
        x_ref[0], w1_ref[...],
        dimension_numbers=(((0,), (0,)), ((), ())),
        precision=_PREC, preferred_element_type=jnp.float32)
    ns = jnp.floor(proj + b1_ref[...][None, :])
    p = ns.reshape(NUM_CHUNKS, CHUNK_FLAT).astype(jnp.bfloat16)
    r = jnp.dot(p, wcat_ref[...], preferred_element_type=jnp.float32)
    u = r[:, :EMBED_DIM] + r[:, EMBED_DIM:2 * EMBED_DIM]
    v = r[:, 2 * EMBED_DIM:3 * EMBED_DIM] + r[:, 3 * EMBED_DIM:]
    out_ref[0] = jnp.floor(u[:NUM_TOKENS] + v[1:NUM_TOKENS + 1]
                           + b2_ref[...][None, :])


def _imap3(b):
    z = b - b
    return (b, z, z)


def _imap2(b):
    z = b - b
    return (z, z)


def _imap1(b):
    return (b - b,)


@functools.partial(jax.jit, static_argnames=())
def kernel(x, W1, b1, W2, b2):
    xf = x.astype(jnp.float32)
    w1f = W1.astype(jnp.float32)
    b1f = b1.astype(jnp.float32)
    b2f = b2.astype(jnp.float32)
    w2f = W2.astype(jnp.float32).reshape(EMBED_DIM, WINDOW_SIZE, EMBED_DIM)
    w2p = jnp.transpose(w2f, (1, 0, 2))
    wtop = w2p[:STEP_SIZE].reshape(CHUNK_FLAT, EMBED_DIM)
    wbot = w2p[STEP_SIZE:].reshape(CHUNK_FLAT, EMBED_DIM)
    wth = wtop.astype(jnp.bfloat16)
    wtl = (wtop - wth.astype(jnp.float32)).astype(jnp.bfloat16)
    wbh = wbot.astype(jnp.bfloat16)
    wbl = (wbot - wbh.astype(jnp.float32)).astype(jnp.bfloat16)
    wcat = jnp.concatenate([wth, wtl, wbh, wbl], axis=1)

    tokens = pl.pallas_call(
        _fused_kernel,
        grid=(BATCH,),
        in_specs=[
            pl.BlockSpec((1, V, SAMPLES), _imap3),
            pl.BlockSpec((V, EMBED_DIM), _imap2),
            pl.BlockSpec((EMBED_DIM,), _imap1),
            pl.BlockSpec((CHUNK_FLAT, 4 * EMBED_DIM), _imap2),
            pl.BlockSpec((EMBED_DIM,), _imap1),
        ],
        out_specs=pl.BlockSpec((1, NUM_TOKENS, EMBED_DIM), _imap3),
        out_shape=jax.ShapeDtypeStruct((BATCH, NUM_TOKENS, EMBED_DIM),
                                       jnp.float32),
    )(xf, w1f, b1f, wcat, b2f)

    return tokens.astype(jnp.float64)


# speedup vs baseline: 168.9349x; 1.3693x over previous
import functools

import jax
import jax.numpy as jnp
from jax.experimental import pallas as pl

BATCH, V, SAMPLES = 32, 64, 4096
EMBED_DIM = 128
WINDOW_SIZE = 64
STEP_SIZE = 32
NUM_TOKENS = (SAMPLES - WINDOW_SIZE) // STEP_SIZE
NUM_CHUNKS = SAMPLES // STEP_SIZE
CHUNK_FLAT = STEP_SIZE * EMBED_DIM

_PREC = jax.lax.Precision.HIGHEST


def _fused_kernel(x_ref, w1_ref, b1_ref, wcat_ref, b2_ref, out_ref):
    xf = x_ref[0]
    xh = xf.astype(jnp.bfloat16)
    xl = (xf - xh.astype(jnp.float32)).astype(jnp.bfloat16)
    x2 = jnp.concatenate([xh, xl], axis=0)
    r1 = jax.lax.dot_general(
        x2, w1_ref[...],
        dimension_numbers=(((0,), (0,)), ((), ())),
        preferred_element_type=jnp.float32)
    proj = r1[:, :EMBED_DIM] + r1[:, EMBED_DIM:]
    ns = jnp.floor(proj + b1_ref[...][None, :])
    p = ns.astype(jnp.bfloat16).reshape(NUM_CHUNKS, CHUNK_FLAT)
    r = jnp.dot(p, wcat_ref[...], preferred_element_type=jnp.float32)
    u = r[:, :EMBED_DIM] + r[:, EMBED_DIM:2 * EMBED_DIM]
    v = r[:, 2 * EMBED_DIM:3 * EMBED_DIM] + r[:, 3 * EMBED_DIM:]
    out_ref[0] = jnp.floor(u[:NUM_TOKENS] + v[1:NUM_TOKENS + 1]
                           + b2_ref[...][None, :])


def _imap3(b):
    z = b - b
    return (b, z, z)


def _imap2(b):
    z = b - b
    return (z, z)


def _imap1(b):
    return (b - b,)


@functools.partial(jax.jit, static_argnames=())
def kernel(x, W1, b1, W2, b2):
    xf = x.astype(jnp.float32)
    w1f = W1.astype(jnp.float32)
    b1f = b1.astype(jnp.float32)
    b2f = b2.astype(jnp.float32)
    w1h = w1f.astype(jnp.bfloat16)
    w1l = (w1f - w1h.astype(jnp.float32)).astype(jnp.bfloat16)
    zeros = jnp.zeros_like(w1h)
    w1cat = jnp.concatenate(
        [jnp.concatenate([w1h, w1l], axis=1),
         jnp.concatenate([w1h, zeros], axis=1)], axis=0)
    w2f = W2.astype(jnp.float32).reshape(EMBED_DIM, WINDOW_SIZE, EMBED_DIM)
    w2p = jnp.transpose(w2f, (1, 0, 2))
    wtop = w2p[:STEP_SIZE].reshape(CHUNK_FLAT, EMBED_DIM)
    wbot = w2p[STEP_SIZE:].reshape(CHUNK_FLAT, EMBED_DIM)
    wth = wtop.astype(jnp.bfloat16)
    wtl = (wtop - wth.astype(jnp.float32)).astype(jnp.bfloat16)
    wbh = wbot.astype(jnp.bfloat16)
    wbl = (wbot - wbh.astype(jnp.float32)).astype(jnp.bfloat16)
    wcat = jnp.concatenate([wth, wtl, wbh, wbl], axis=1)

    tokens = pl.pallas_call(
        _fused_kernel,
        grid=(BATCH,),
        in_specs=[
            pl.BlockSpec((1, V, SAMPLES), _imap3),
            pl.BlockSpec((2 * V, 2 * EMBED_DIM), _imap2),
            pl.BlockSpec((EMBED_DIM,), _imap1),
            pl.BlockSpec((CHUNK_FLAT, 4 * EMBED_DIM), _imap2),
            pl.BlockSpec((EMBED_DIM,), _imap1),
        ],
        out_specs=pl.BlockSpec((1, NUM_TOKENS, EMBED_DIM), _imap3),
        out_shape=jax.ShapeDtypeStruct((BATCH, NUM_TOKENS, EMBED_DIM),
                                       jnp.float32),
    )(xf, w1cat, b1f, wcat, b2f)

    return tokens.astype(jnp.float64)


# speedup vs baseline: 222.5611x; 1.3174x over previous
import functools

import jax
import jax.numpy as jnp
from jax.experimental import pallas as pl

BATCH, V, SAMPLES = 32, 64, 4096
EMBED_DIM = 128
WINDOW_SIZE = 64
STEP_SIZE = 32
NUM_TOKENS = (SAMPLES - WINDOW_SIZE) // STEP_SIZE
NUM_CHUNKS = SAMPLES // STEP_SIZE
CHUNK_FLAT = STEP_SIZE * EMBED_DIM

_PREC = jax.lax.Precision.HIGHEST


def _split_hi_lo(a):
    bits = jax.lax.bitcast_convert_type(a, jnp.uint32)
    hi = jax.lax.bitcast_convert_type(
        jax.lax.bitwise_and(bits, jnp.uint32(0xFFFF0000)), jnp.float32)
    return hi.astype(jnp.bfloat16), (a - hi).astype(jnp.bfloat16)


def _fused_kernel(x_ref, w1_ref, b1_ref, wcat_ref, b2_ref, out_ref):
    xh, xl = _split_hi_lo(x_ref[0])
    x3 = jnp.concatenate([xh, xl, xh], axis=0)
    proj = jax.lax.dot_general(
        x3, w1_ref[...],
        dimension_numbers=(((0,), (0,)), ((), ())),
        preferred_element_type=jnp.float32)
    ns = jnp.floor(proj + b1_ref[...][None, :])
    p = ns.astype(jnp.bfloat16).reshape(NUM_CHUNKS, CHUNK_FLAT)
    r = jnp.dot(p, wcat_ref[...], preferred_element_type=jnp.float32)
    u = r[:, :EMBED_DIM]
    v = r[:, EMBED_DIM:]
    out_ref[0] = jnp.floor(u[:NUM_TOKENS] + v[1:NUM_TOKENS + 1]
                           + b2_ref[...][None, :])


def _imap3(b):
    z = b - b
    return (b, z, z)


def _imap2(b):
    z = b - b
    return (z, z)


def _imap1(b):
    return (b - b,)


@functools.partial(jax.jit, static_argnames=())
def kernel(x, W1, b1, W2, b2):
    xf = x.astype(jnp.float32)
    w1f = W1.astype(jnp.float32)
    b1f = b1.astype(jnp.float32)
    b2f = b2.astype(jnp.float32)
    w1bits = jax.lax.bitcast_convert_type(w1f, jnp.uint32)
    w1hf = jax.lax.bitcast_convert_type(
        jax.lax.bitwise_and(w1bits, jnp.uint32(0xFFFF0000)), jnp.float32)
    w1h = w1hf.astype(jnp.bfloat16)
    w1l = (w1f - w1hf).astype(jnp.bfloat16)
    w1cat = jnp.concatenate([w1h, w1h, w1l], axis=0)
    w2f = W2.astype(jnp.float32).reshape(EMBED_DIM, WINDOW_SIZE, EMBED_DIM)
    w2p = jnp.transpose(w2f, (1, 0, 2))
    wtop = w2p[:STEP_SIZE].reshape(CHUNK_FLAT, EMBED_DIM)
    wbot = w2p[STEP_SIZE:].reshape(CHUNK_FLAT, EMBED_DIM)
    wcat = jnp.concatenate([wtop.astype(jnp.bfloat16),
                            wbot.astype(jnp.bfloat16)], axis=1)

    tokens = pl.pallas_call(
        _fused_kernel,
        grid=(BATCH,),
        in_specs=[
            pl.BlockSpec((1, V, SAMPLES), _imap3),
            pl.BlockSpec((3 * V, EMBED_DIM), _imap2),
            pl.BlockSpec((EMBED_DIM,), _imap1),
            pl.BlockSpec((CHUNK_FLAT, 2 * EMBED_DIM), _imap2),
            pl.BlockSpec((EMBED_DIM,), _imap1),
        ],
        out_specs=pl.BlockSpec((1, NUM_TOKENS, EMBED_DIM), _imap3),
        out_shape=jax.ShapeDtypeStruct((BATCH, NUM_TOKENS, EMBED_DIM),
                                       jnp.float32),
    )(xf, w1cat, b1f, wcat, b2f)

    return tokens.astype(jnp.float64)


# speedup vs baseline: 254.9399x; 1.1455x over previous
import functools

import jax
import jax.numpy as jnp
from jax.experimental import pallas as pl

BATCH, V, SAMPLES = 32, 64, 4096
EMBED_DIM = 128
WINDOW_SIZE = 64
STEP_SIZE = 32
NUM_TOKENS = (SAMPLES - WINDOW_SIZE) // STEP_SIZE
NUM_CHUNKS = SAMPLES // STEP_SIZE
CHUNK_FLAT = STEP_SIZE * EMBED_DIM

_PREC = jax.lax.Precision.HIGHEST


def _split_hi_lo(a):
    bits = jax.lax.bitcast_convert_type(a, jnp.uint32)
    hi = jax.lax.bitcast_convert_type(
        jax.lax.bitwise_and(bits, jnp.uint32(0xFFFF0000)), jnp.float32)
    return hi.astype(jnp.bfloat16), (a - hi).astype(jnp.bfloat16)


ROWS = 4


def _fused_kernel(x_ref, w1_ref, b1_ref, wcat_ref, b2_ref, out_ref):
    xh, xl = _split_hi_lo(x_ref[...])
    cols = [jnp.concatenate([xh[r], xl[r], xh[r], xl[r]], axis=0)
            for r in range(ROWS)]
    lhs = jnp.concatenate(cols, axis=1)
    proj = jax.lax.dot_general(
        lhs, w1_ref[...],
        dimension_numbers=(((0,), (0,)), ((), ())),
        preferred_element_type=jnp.float32)
    ns = jnp.floor(proj + b1_ref[...][None, :])
    p = ns.astype(jnp.bfloat16).reshape(ROWS * NUM_CHUNKS, CHUNK_FLAT)
    r = jnp.dot(p, wcat_ref[...], preferred_element_type=jnp.float32)
    u = r[:, :EMBED_DIM]
    v = r[:, EMBED_DIM:]
    b2v = b2_ref[...][None, :]
    for r_i in range(ROWS):
        base = r_i * NUM_CHUNKS
        out_ref[r_i] = jnp.floor(u[base:base + NUM_TOKENS]
                                 + v[base + 1:base + NUM_TOKENS + 1] + b2v)


def _imap3(b):
    z = b - b
    return (b, z, z)


def _imap2(b):
    z = b - b
    return (z, z)


def _imap1(b):
    return (b - b,)


@functools.partial(jax.jit, static_argnames=())
def kernel(x, W1, b1, W2, b2):
    xf = x.astype(jnp.float32)
    w1f = W1.astype(jnp.float32)
    b1f = b1.astype(jnp.float32)
    b2f = b2.astype(jnp.float32)
    w1bits = jax.lax.bitcast_convert_type(w1f, jnp.uint32)
    w1hf = jax.lax.bitcast_convert_type(
        jax.lax.bitwise_and(w1bits, jnp.uint32(0xFFFF0000)), jnp.float32)
    w1h = w1hf.astype(jnp.bfloat16)
    w1l = (w1f - w1hf).astype(jnp.bfloat16)
    w1cat = jnp.concatenate([w1h, w1h, w1l, w1l], axis=0)
    w2b = W2.astype(jnp.bfloat16).reshape(EMBED_DIM, WINDOW_SIZE, EMBED_DIM)
    w2p = jnp.transpose(w2b, (1, 0, 2))
    wtop = w2p[:STEP_SIZE].reshape(CHUNK_FLAT, EMBED_DIM)
    wbot = w2p[STEP_SIZE:].reshape(CHUNK_FLAT, EMBED_DIM)
    wcat = jnp.concatenate([wtop, wbot], axis=1)

    tokens = pl.pallas_call(
        _fused_kernel,
        grid=(BATCH // ROWS,),
        in_specs=[
            pl.BlockSpec((ROWS, V, SAMPLES), _imap3),
            pl.BlockSpec((4 * V, EMBED_DIM), _imap2),
            pl.BlockSpec((EMBED_DIM,), _imap1),
            pl.BlockSpec((CHUNK_FLAT, 2 * EMBED_DIM), _imap2),
            pl.BlockSpec((EMBED_DIM,), _imap1),
        ],
        out_specs=pl.BlockSpec((ROWS, NUM_TOKENS, EMBED_DIM), _imap3),
        out_shape=jax.ShapeDtypeStruct((BATCH, NUM_TOKENS, EMBED_DIM),
                                       jnp.float32),
    )(xf, w1cat, b1f, wcat, b2f)

    return tokens.astype(jnp.float64)
